# Initial kernel scaffold; baseline (speedup 1.0000x reference)
import jax
import jax.numpy as jnp
from jax.experimental import pallas as pl


def kernel(opacity, deltas, leaves):
    raise NotImplementedError("write your pallas kernel here")


# speedup vs baseline: 138.5175x; 138.5175x over previous
import functools

import jax
import jax.numpy as jnp
from jax import lax
from jax.experimental import pallas as pl
from jax.experimental.pallas import tpu as pltpu
from jax.experimental.pallas import tpu_sc as plsc


def _build(C, N, S):
    L = 16
    NW = 32
    R = C * N
    T = 160
    CW = T * S
    NCH = R // T
    G = T // L

    mesh = plsc.VectorSubcoreMesh(core_axis_name="c", subcore_axis_name="s")

    @functools.partial(
        pl.kernel,
        out_type=jax.ShapeDtypeStruct((R,), jnp.float32),
        mesh=mesh,
        scratch_types=[
            pltpu.VMEM((N,), jnp.float32),
            pltpu.VMEM((CW,), jnp.int32),
            pltpu.VMEM((CW,), jnp.float32),
            pltpu.VMEM((T,), jnp.float32),
        ],
        compiler_params=pltpu.CompilerParams(needs_layout_passes=False),
    )
    def k(op_hbm, deltas_hbm, leaves_hbm, out_hbm, table_v, lv_v, dv_v, out_v):
        wid = lax.axis_index("s") * 2 + lax.axis_index("c")
        pltpu.sync_copy(op_hbm, table_v)
        ray_off = lax.iota(jnp.int32, L) * S
        nch = (NCH - wid + NW - 1) // NW

        def chunk_body(i, _):
            c = wid + i * NW
            base = c * T
            pltpu.sync_copy(leaves_hbm.at[pl.ds(base * S, CW)], lv_v)
            pltpu.sync_copy(deltas_hbm.at[pl.ds(base * S, CW)], dv_v)
            for g in range(G):
                idx0 = ray_off + (g * L * S)

                def s_body(s, acc):
                    idx = idx0 + s
                    leaf = plsc.load_gather(lv_v, [idx])
                    dv = plsc.load_gather(dv_v, [idx])
                    op = plsc.load_gather(table_v, [leaf])
                    return acc + op * dv

                acc = lax.fori_loop(0, S, s_body, jnp.zeros((L,), jnp.float32))
                out_v[pl.ds(g * L, L)] = 1.0 - jnp.exp(-acc)
            pltpu.sync_copy(out_v, out_hbm.at[pl.ds(base, T)])
            return 0

        lax.fori_loop(0, nch, chunk_body, 0)

    return k


def kernel(opacity, deltas, leaves):
    C, N, S = deltas.shape
    k = _build(C, N, S)
    out = k(opacity, deltas.reshape(-1), leaves.reshape(-1))
    return out.reshape(C, N)


# speedup vs baseline: 172.8398x; 1.2478x over previous
import functools

import jax
import jax.numpy as jnp
from jax import lax
from jax.experimental import pallas as pl
from jax.experimental.pallas import tpu as pltpu
from jax.experimental.pallas import tpu_sc as plsc


def _build(C, N, S):
    L = 16
    NW = 32
    R = C * N
    T = 160
    CW = T * S
    NCH = R // T
    G = T // L

    mesh = plsc.VectorSubcoreMesh(core_axis_name="c", subcore_axis_name="s")

    @functools.partial(
        pl.kernel,
        out_type=jax.ShapeDtypeStruct((R,), jnp.float32),
        mesh=mesh,
        scratch_types=[
            pltpu.VMEM((N,), jnp.float32),
            pltpu.VMEM((2 * CW,), jnp.int32),
            pltpu.VMEM((2 * CW,), jnp.float32),
            pltpu.VMEM((2 * T,), jnp.float32),
            pltpu.SemaphoreType.DMA,
            pltpu.SemaphoreType.DMA,
        ],
        compiler_params=pltpu.CompilerParams(needs_layout_passes=False),
    )
    def k(op_hbm, deltas_hbm, leaves_hbm, out_hbm, table_v, lv_v, dv_v, out_v,
          sem_in, sem_out):
        wid = lax.axis_index("s") * 2 + lax.axis_index("c")
        nch = (NCH - wid + NW - 1) // NW
        ray_off = lax.iota(jnp.int32, L) * S

        def start_in(c, badd):
            pltpu.async_copy(
                leaves_hbm.at[pl.ds(c * CW, CW)], lv_v.at[pl.ds(badd, CW)],
                sem_in)
            pltpu.async_copy(
                deltas_hbm.at[pl.ds(c * CW, CW)], dv_v.at[pl.ds(badd, CW)],
                sem_in)

        def wait_in():
            pltpu.make_async_copy(
                leaves_hbm.at[pl.ds(0, CW)], lv_v.at[pl.ds(0, CW)],
                sem_in).wait()
            pltpu.make_async_copy(
                deltas_hbm.at[pl.ds(0, CW)], dv_v.at[pl.ds(0, CW)],
                sem_in).wait()

        def wait_out():
            pltpu.make_async_copy(
                out_v.at[pl.ds(0, T)], out_hbm.at[pl.ds(0, T)],
                sem_out).wait()

        start_in(wid, 0)
        pltpu.sync_copy(op_hbm, table_v)

        def chunk_body(i, _):
            badd = lax.rem(i, 2) * CW
            obase = lax.rem(i, 2) * T
            c = wid + i * NW

            @pl.when(i + 1 < nch)
            def _():
                start_in(c + NW, CW - badd)

            wait_in()

            @pl.when(i >= 2)
            def _():
                wait_out()

            def group_body(g, _):
                coff = badd + g * (L * S)
                accs = [jnp.zeros((L,), jnp.float32) for _ in range(4)]
                for s in range(S):
                    idx = ray_off + (coff + s)
                    leaf = plsc.load_gather(lv_v, [idx])
                    dv = plsc.load_gather(dv_v, [idx])
                    op = plsc.load_gather(table_v, [leaf])
                    accs[s % 4] = accs[s % 4] + op * dv
                acc = (accs[0] + accs[1]) + (accs[2] + accs[3])
                out_v[pl.ds(obase + g * L, L)] = 1.0 - jnp.exp(-acc)
                return 0

            lax.fori_loop(0, G, group_body, 0)
            pltpu.async_copy(
                out_v.at[pl.ds(obase, T)], out_hbm.at[pl.ds(c * T, T)],
                sem_out)
            return 0

        lax.fori_loop(0, nch, chunk_body, 0)
        wait_out()
        wait_out()

    return k


def kernel(opacity, deltas, leaves):
    C, N, S = deltas.shape
    k = _build(C, N, S)
    out = k(opacity, deltas.reshape(-1), leaves.reshape(-1))
    return out.reshape(C, N)
